# baseline (device time: 40350 ns/iter reference)
import jax
import jax.numpy as jnp
from jax import lax
from jax.experimental import pallas as pl
from jax.experimental.pallas import tpu as pltpu

B, S, H, DH, DR = 2, 256, 16, 64, 32
D = 1024
DC_SH = 64
BS = B * S
SCALE = (DH + DR) ** -0.5


def kernel(x, Wdkv, Wuk, Wuv, Wq, Wqr, Wkr, Wo):
    def body(
        x_ref, wdkv_ref, wuk_ref, wuv_ref, wq_ref, wqr_ref, wkr_ref, wo_ref,
        out_ref,
        c_send, c_recv, wuk_send, wuk_recv, wuv_send, wuv_recv,
        o_scratch, send_sems, recv_sems,
    ):
        my_x = lax.axis_index("x")
        my_y = lax.axis_index("y")
        peer = (1 - my_x, my_y)

        barrier = pltpu.get_barrier_semaphore()
        pl.semaphore_signal(
            barrier, inc=1, device_id=peer, device_id_type=pl.DeviceIdType.MESH
        )
        pl.semaphore_wait(barrier, 1)

        xb = x_ref[...].reshape(BS, D).astype(jnp.bfloat16)

        c_send[...] = lax.dot(
            xb, wdkv_ref[...].astype(jnp.bfloat16),
            preferred_element_type=jnp.float32,
        ).astype(jnp.bfloat16)
        wuk_send[...] = wuk_ref[...].astype(jnp.bfloat16)
        wuv_send[...] = wuv_ref[...].astype(jnp.bfloat16)

        rdmas = []
        for i, (src, dst) in enumerate(
            [(c_send, c_recv), (wuk_send, wuk_recv), (wuv_send, wuv_recv)]
        ):
            rdma = pltpu.make_async_remote_copy(
                src_ref=src, dst_ref=dst,
                send_sem=send_sems.at[i], recv_sem=recv_sems.at[i],
                device_id=peer, device_id_type=pl.DeviceIdType.MESH,
            )
            rdma.start()
            rdmas.append(rdma)

        q2d = lax.dot(
            xb, wq_ref[...].astype(jnp.bfloat16),
            preferred_element_type=jnp.float32,
        ).astype(jnp.bfloat16)
        qr2d = lax.dot(
            xb, wqr_ref[...].astype(jnp.bfloat16),
            preferred_element_type=jnp.float32,
        ).astype(jnp.bfloat16)
        kr2d = lax.dot(
            xb, wkr_ref[...].astype(jnp.bfloat16),
            preferred_element_type=jnp.float32,
        ).astype(jnp.bfloat16)

        c_loc = c_send[...]
        k_part = lax.dot(c_loc, wuk_send[...], preferred_element_type=jnp.float32)
        v_part = lax.dot(c_loc, wuv_send[...], preferred_element_type=jnp.float32)

        for rdma in rdmas:
            rdma.wait()

        k2d = (
            k_part
            + lax.dot(c_recv[...], wuk_recv[...], preferred_element_type=jnp.float32)
        ).astype(jnp.bfloat16)
        v2d = (
            v_part
            + lax.dot(c_recv[...], wuv_recv[...], preferred_element_type=jnp.float32)
        ).astype(jnp.bfloat16)

        tt = (((1,), (1,)), ((), ()))
        for b in range(B):
            rows = slice(b * S, (b + 1) * S)
            kr_b = kr2d[rows, :]
            for h in range(H):
                cols = slice(h * DH, (h + 1) * DH)
                q_bh = q2d[rows, cols]
                k_bh = k2d[rows, cols]
                qr_bh = qr2d[rows, h * DR:(h + 1) * DR]
                s = lax.dot_general(
                    q_bh, k_bh, tt, preferred_element_type=jnp.float32
                ) + lax.dot_general(
                    qr_bh, kr_b, tt, preferred_element_type=jnp.float32
                )
                s = s * SCALE
                m = jnp.max(s, axis=1, keepdims=True)
                p = jnp.exp(s - m)
                p = p / jnp.sum(p, axis=1, keepdims=True)
                o_scratch[rows, cols] = lax.dot(
                    p.astype(jnp.bfloat16), v2d[rows, cols],
                    preferred_element_type=jnp.float32,
                )

        out_ref[...] = lax.dot(
            o_scratch[...].astype(jnp.bfloat16), wo_ref[...].astype(jnp.bfloat16),
            preferred_element_type=jnp.float32,
        ).reshape(B, S, D)

    return pl.pallas_call(
        body,
        out_shape=jax.ShapeDtypeStruct((B, S, D), jnp.float32),
        in_specs=[pl.BlockSpec(memory_space=pltpu.VMEM)] * 8,
        out_specs=pl.BlockSpec(memory_space=pltpu.VMEM),
        scratch_shapes=[
            pltpu.VMEM((BS, DC_SH), jnp.bfloat16),
            pltpu.VMEM((BS, DC_SH), jnp.bfloat16),
            pltpu.VMEM((DC_SH, D), jnp.bfloat16),
            pltpu.VMEM((DC_SH, D), jnp.bfloat16),
            pltpu.VMEM((DC_SH, D), jnp.bfloat16),
            pltpu.VMEM((DC_SH, D), jnp.bfloat16),
            pltpu.VMEM((BS, D), jnp.float32),
            pltpu.SemaphoreType.DMA((3,)),
            pltpu.SemaphoreType.DMA((3,)),
        ],
        compiler_params=pltpu.CompilerParams(collective_id=0),
    )(x, Wdkv, Wuk, Wuv, Wq, Wqr, Wkr, Wo)


# device time: 33247 ns/iter; 1.2136x vs baseline; 1.2136x over previous
import jax
import jax.numpy as jnp
from jax import lax
from jax.experimental import pallas as pl
from jax.experimental.pallas import tpu as pltpu

B, S, H, DH, DR = 2, 256, 16, 64, 32
D = 1024
DC_SH = 64
BS = B * S
SCALE = (DH + DR) ** -0.5


def kernel(x, Wdkv, Wuk, Wuv, Wq, Wqr, Wkr, Wo):
    def body(
        x_ref, wdkv_ref, wuk_ref, wuv_ref, wq_hbm, wqr_hbm, wkr_ref, wo_hbm,
        out_ref,
        c_send, c_recv, wuk_send, wuk_recv, wuv_send, wuv_recv,
        wq_vmem, wqr_vmem, wo_vmem,
        o_scratch, send_sems, recv_sems, dma_sems,
    ):
        my_x = lax.axis_index("x")
        my_y = lax.axis_index("y")
        peer = (1 - my_x, my_y)

        wq_dma = pltpu.make_async_copy(wq_hbm, wq_vmem, dma_sems.at[0])
        wqr_dma = pltpu.make_async_copy(wqr_hbm, wqr_vmem, dma_sems.at[1])
        wo_dma = pltpu.make_async_copy(wo_hbm, wo_vmem, dma_sems.at[2])
        wq_dma.start()
        wqr_dma.start()
        wo_dma.start()

        barrier = pltpu.get_barrier_semaphore()
        pl.semaphore_signal(
            barrier, inc=1, device_id=peer, device_id_type=pl.DeviceIdType.MESH
        )
        pl.semaphore_wait(barrier, 1)

        wuk_send[...] = wuk_ref[...].astype(jnp.bfloat16)
        wuv_send[...] = wuv_ref[...].astype(jnp.bfloat16)
        rdmas = []
        for i, (src, dst) in enumerate(
            [(wuk_send, wuk_recv), (wuv_send, wuv_recv)]
        ):
            rdma = pltpu.make_async_remote_copy(
                src_ref=src, dst_ref=dst,
                send_sem=send_sems.at[i], recv_sem=recv_sems.at[i],
                device_id=peer, device_id_type=pl.DeviceIdType.MESH,
            )
            rdma.start()
            rdmas.append(rdma)

        xb = x_ref[...].reshape(BS, D).astype(jnp.bfloat16)
        c_send[...] = lax.dot(
            xb, wdkv_ref[...].astype(jnp.bfloat16),
            preferred_element_type=jnp.float32,
        ).astype(jnp.bfloat16)
        c_rdma = pltpu.make_async_remote_copy(
            src_ref=c_send, dst_ref=c_recv,
            send_sem=send_sems.at[2], recv_sem=recv_sems.at[2],
            device_id=peer, device_id_type=pl.DeviceIdType.MESH,
        )
        c_rdma.start()
        rdmas.append(c_rdma)

        wq_dma.wait()
        q2d = lax.dot(
            xb, wq_vmem[...].astype(jnp.bfloat16),
            preferred_element_type=jnp.float32,
        ).astype(jnp.bfloat16)
        wqr_dma.wait()
        qr2d = lax.dot(
            xb, wqr_vmem[...].astype(jnp.bfloat16),
            preferred_element_type=jnp.float32,
        ).astype(jnp.bfloat16)
        kr2d = lax.dot(
            xb, wkr_ref[...].astype(jnp.bfloat16),
            preferred_element_type=jnp.float32,
        ).astype(jnp.bfloat16)

        c_loc = c_send[...]
        k_part = lax.dot(c_loc, wuk_send[...], preferred_element_type=jnp.float32)
        v_part = lax.dot(c_loc, wuv_send[...], preferred_element_type=jnp.float32)

        for rdma in rdmas:
            rdma.wait()

        k2d = (
            k_part
            + lax.dot(c_recv[...], wuk_recv[...], preferred_element_type=jnp.float32)
        ).astype(jnp.bfloat16)
        v2d = (
            v_part
            + lax.dot(c_recv[...], wuv_recv[...], preferred_element_type=jnp.float32)
        ).astype(jnp.bfloat16)

        tt = (((1,), (1,)), ((), ()))
        for b in range(B):
            rows = slice(b * S, (b + 1) * S)
            kr_b = kr2d[rows, :]
            for h in range(H):
                cols = slice(h * DH, (h + 1) * DH)
                s = lax.dot_general(
                    q2d[rows, cols], k2d[rows, cols], tt,
                    preferred_element_type=jnp.float32,
                ) + lax.dot_general(
                    qr2d[rows, h * DR:(h + 1) * DR], kr_b, tt,
                    preferred_element_type=jnp.float32,
                )
                p = jnp.exp(s * SCALE)
                o_bh = lax.dot(
                    p.astype(jnp.bfloat16), v2d[rows, cols],
                    preferred_element_type=jnp.float32,
                )
                o_scratch[rows, cols] = o_bh / jnp.sum(p, axis=1, keepdims=True)

        wo_dma.wait()
        out_ref[...] = lax.dot(
            o_scratch[...].astype(jnp.bfloat16), wo_vmem[...].astype(jnp.bfloat16),
            preferred_element_type=jnp.float32,
        ).reshape(B, S, D)

    vmem = pl.BlockSpec(memory_space=pltpu.VMEM)
    hbm = pl.BlockSpec(memory_space=pl.ANY)
    return pl.pallas_call(
        body,
        out_shape=jax.ShapeDtypeStruct((B, S, D), jnp.float32),
        in_specs=[vmem, vmem, vmem, vmem, hbm, hbm, vmem, hbm],
        out_specs=vmem,
        scratch_shapes=[
            pltpu.VMEM((BS, DC_SH), jnp.bfloat16),
            pltpu.VMEM((BS, DC_SH), jnp.bfloat16),
            pltpu.VMEM((DC_SH, D), jnp.bfloat16),
            pltpu.VMEM((DC_SH, D), jnp.bfloat16),
            pltpu.VMEM((DC_SH, D), jnp.bfloat16),
            pltpu.VMEM((DC_SH, D), jnp.bfloat16),
            pltpu.VMEM((D, D), jnp.float32),
            pltpu.VMEM((D, H * DR), jnp.float32),
            pltpu.VMEM((D, D), jnp.float32),
            pltpu.VMEM((BS, D), jnp.float32),
            pltpu.SemaphoreType.DMA((3,)),
            pltpu.SemaphoreType.DMA((3,)),
            pltpu.SemaphoreType.DMA((3,)),
        ],
        compiler_params=pltpu.CompilerParams(collective_id=0),
    )(x, Wdkv, Wuk, Wuv, Wq, Wqr, Wkr, Wo)


# device time: 31740 ns/iter; 1.2713x vs baseline; 1.0475x over previous
import jax
import jax.numpy as jnp
from jax import lax
from jax.experimental import pallas as pl
from jax.experimental.pallas import tpu as pltpu

B, S, H, DH, DR = 2, 256, 16, 64, 32
D = 1024
DC_SH = 64
BS = B * S
DP = 128
SCALE = (DH + DR) ** -0.5


def kernel(x, Wdkv, Wuk, Wuv, Wq, Wqr, Wkr, Wo):
    def body(
        x_ref, wdkv_ref, wuk_ref, wuv_ref, wq_hbm, wqr_hbm, wkr_ref, wo_hbm,
        out_ref,
        c_send, c_recv, wuk_send, wuk_recv, wuv_send, wuv_recv,
        wq_vmem, wqr_vmem, wo_vmem,
        o_scratch, qcat, kcat, send_sems, recv_sems, dma_sems,
    ):
        my_x = lax.axis_index("x")
        my_y = lax.axis_index("y")
        peer = (1 - my_x, my_y)

        wq_dma = pltpu.make_async_copy(wq_hbm, wq_vmem, dma_sems.at[0])
        wqr_dma = pltpu.make_async_copy(wqr_hbm, wqr_vmem, dma_sems.at[1])
        wo_dma = pltpu.make_async_copy(wo_hbm, wo_vmem, dma_sems.at[2])
        wq_dma.start()
        wqr_dma.start()
        wo_dma.start()

        barrier = pltpu.get_barrier_semaphore()
        pl.semaphore_signal(
            barrier, inc=1, device_id=peer, device_id_type=pl.DeviceIdType.MESH
        )
        pl.semaphore_wait(barrier, 1)

        wuk_send[...] = wuk_ref[...].astype(jnp.bfloat16)
        wuv_send[...] = wuv_ref[...].astype(jnp.bfloat16)
        rdmas = []
        for i, (src, dst) in enumerate(
            [(wuk_send, wuk_recv), (wuv_send, wuv_recv)]
        ):
            rdma = pltpu.make_async_remote_copy(
                src_ref=src, dst_ref=dst,
                send_sem=send_sems.at[i], recv_sem=recv_sems.at[i],
                device_id=peer, device_id_type=pl.DeviceIdType.MESH,
            )
            rdma.start()
            rdmas.append(rdma)

        xb = x_ref[...].reshape(BS, D).astype(jnp.bfloat16)
        c_send[...] = lax.dot(
            xb, wdkv_ref[...].astype(jnp.bfloat16),
            preferred_element_type=jnp.float32,
        ).astype(jnp.bfloat16)
        c_rdma = pltpu.make_async_remote_copy(
            src_ref=c_send, dst_ref=c_recv,
            send_sem=send_sems.at[2], recv_sem=recv_sems.at[2],
            device_id=peer, device_id_type=pl.DeviceIdType.MESH,
        )
        c_rdma.start()
        rdmas.append(c_rdma)

        wq_dma.wait()
        q2d = (lax.dot(
            xb, wq_vmem[...].astype(jnp.bfloat16),
            preferred_element_type=jnp.float32,
        ) * SCALE).astype(jnp.bfloat16)
        wqr_dma.wait()
        qr2d = (lax.dot(
            xb, wqr_vmem[...].astype(jnp.bfloat16),
            preferred_element_type=jnp.float32,
        ) * SCALE).astype(jnp.bfloat16)
        kr2d = lax.dot(
            xb, wkr_ref[...].astype(jnp.bfloat16),
            preferred_element_type=jnp.float32,
        ).astype(jnp.bfloat16)

        c_loc = c_send[...]
        k_part = lax.dot(c_loc, wuk_send[...], preferred_element_type=jnp.float32)
        v_part = lax.dot(c_loc, wuv_send[...], preferred_element_type=jnp.float32)

        for rdma in rdmas:
            rdma.wait()

        k2d = (
            k_part
            + lax.dot(c_recv[...], wuk_recv[...], preferred_element_type=jnp.float32)
        ).astype(jnp.bfloat16)
        v2d = (
            v_part
            + lax.dot(c_recv[...], wuv_recv[...], preferred_element_type=jnp.float32)
        ).astype(jnp.bfloat16)

        zpad = jnp.zeros((BS, DP - DH - DR), jnp.bfloat16)
        for h in range(H):
            base = h * DP
            qcat[:, base:base + DH] = q2d[:, h * DH:(h + 1) * DH]
            qcat[:, base + DH:base + DH + DR] = qr2d[:, h * DR:(h + 1) * DR]
            qcat[:, base + DH + DR:base + DP] = zpad
            kcat[:, base:base + DH] = k2d[:, h * DH:(h + 1) * DH]
            kcat[:, base + DH:base + DH + DR] = kr2d
            kcat[:, base + DH + DR:base + DP] = zpad

        tt = (((1,), (1,)), ((), ()))
        for b in range(B):
            rows = slice(b * S, (b + 1) * S)
            for h in range(H):
                cols = slice(h * DH, (h + 1) * DH)
                pcols = slice(h * DP, (h + 1) * DP)
                s = lax.dot_general(
                    qcat[rows, pcols], kcat[rows, pcols], tt,
                    preferred_element_type=jnp.float32,
                )
                p = jnp.exp(s)
                o_bh = lax.dot(
                    p.astype(jnp.bfloat16), v2d[rows, cols],
                    preferred_element_type=jnp.float32,
                )
                o_scratch[rows, cols] = o_bh / jnp.sum(p, axis=1, keepdims=True)

        wo_dma.wait()
        out_ref[...] = lax.dot(
            o_scratch[...].astype(jnp.bfloat16), wo_vmem[...].astype(jnp.bfloat16),
            preferred_element_type=jnp.float32,
        ).reshape(B, S, D)

    vmem = pl.BlockSpec(memory_space=pltpu.VMEM)
    hbm = pl.BlockSpec(memory_space=pl.ANY)
    return pl.pallas_call(
        body,
        out_shape=jax.ShapeDtypeStruct((B, S, D), jnp.float32),
        in_specs=[vmem, vmem, vmem, vmem, hbm, hbm, vmem, hbm],
        out_specs=vmem,
        scratch_shapes=[
            pltpu.VMEM((BS, DC_SH), jnp.bfloat16),
            pltpu.VMEM((BS, DC_SH), jnp.bfloat16),
            pltpu.VMEM((DC_SH, D), jnp.bfloat16),
            pltpu.VMEM((DC_SH, D), jnp.bfloat16),
            pltpu.VMEM((DC_SH, D), jnp.bfloat16),
            pltpu.VMEM((DC_SH, D), jnp.bfloat16),
            pltpu.VMEM((D, D), jnp.float32),
            pltpu.VMEM((D, H * DR), jnp.float32),
            pltpu.VMEM((D, D), jnp.float32),
            pltpu.VMEM((BS, D), jnp.float32),
            pltpu.VMEM((BS, H * DP), jnp.bfloat16),
            pltpu.VMEM((BS, H * DP), jnp.bfloat16),
            pltpu.SemaphoreType.DMA((3,)),
            pltpu.SemaphoreType.DMA((3,)),
            pltpu.SemaphoreType.DMA((3,)),
        ],
        compiler_params=pltpu.CompilerParams(collective_id=0),
    )(x, Wdkv, Wuk, Wuv, Wq, Wqr, Wkr, Wo)
